# Initial kernel scaffold; baseline (speedup 1.0000x reference)
import jax
import jax.numpy as jnp
from jax.experimental import pallas as pl


def kernel(x, params):
    raise NotImplementedError("write your pallas kernel here")


# speedup vs baseline: 3.0980x; 3.0980x over previous
import functools

import jax
import jax.numpy as jnp
from jax import lax
from jax.experimental import pallas as pl
from jax.experimental.pallas import tpu as pltpu
from jax.experimental.pallas import tpu_sc as plsc

N = 10000
INPUT_DIM = 128
DIM1 = 64
DIM2 = 32
NBLOCKS = 4
SPACE_DIM = 4
PROP_DIM = 64
K = 16
EPS = 1e-5

SC_NC = 2
SC_NS = 16
SC_NW = SC_NC * SC_NS
ROWS_PER_WORKER = 320
NPAD = SC_NW * ROWS_PER_WORKER
GATHER_ROWS = 8
N_CHUNKS = ROWS_PER_WORKER // GATHER_ROWS

KNN_R = 400
BIG = 1e30


def _knn_body(srb_ref, s_ref, n2c_ref, n2r_ref, idx_ref):
    srb = srb_ref[...].astype(jnp.bfloat16)
    s = s_ref[...].astype(jnp.bfloat16)
    g = lax.dot_general(srb, s, (((1,), (1,)), ((), ())),
                        preferred_element_type=jnp.float32)
    d2 = (n2c_ref[...] + n2r_ref[...]) - 2.0 * g
    iota = lax.broadcasted_iota(jnp.int32, (KNN_R, N), 1)
    lane16 = lax.broadcasted_iota(jnp.int32, (KNN_R, K), 1)

    def body(k, carry):
        d2c, idxa = carry
        m = jnp.min(d2c, axis=1, keepdims=True)
        j = jnp.min(jnp.where(d2c == m, iota, jnp.int32(2**31 - 1)),
                    axis=1, keepdims=True)
        d2c = jnp.where(iota == j, BIG, d2c)
        idxa = jnp.where(lane16 == k, j, idxa)
        return d2c, idxa

    _, idxa = lax.fori_loop(0, K, body, (d2, jnp.zeros((KNN_R, K), jnp.int32)))
    idx_ref[...] = idxa


def _knn_stage(s):
    n2 = jnp.sum(s * s, axis=1)
    grid = (N // KNN_R,)
    return pl.pallas_call(
        _knn_body,
        grid=grid,
        in_specs=[
            pl.BlockSpec((KNN_R, SPACE_DIM), lambda i: (i, 0)),
            pl.BlockSpec((N, SPACE_DIM), lambda i: (0, 0)),
            pl.BlockSpec((KNN_R, 1), lambda i: (i, 0)),
            pl.BlockSpec((1, N), lambda i: (0, 0)),
        ],
        out_specs=pl.BlockSpec((KNN_R, K), lambda i: (i, 0)),
        out_shape=jax.ShapeDtypeStruct((N, K), jnp.int32),
    )(s, s, n2[:, None], n2[None, :])


def _scagg_body(idx_hbm, tab_hbm, s8_hbm, agg_hbm, idx_v, s8_v, buf, out_v, sem):
    wid = lax.axis_index("s") * SC_NC + lax.axis_index("c")
    base = wid * ROWS_PER_WORKER
    pltpu.sync_copy(idx_hbm.at[pl.ds(base * K, ROWS_PER_WORKER * K)], idx_v)
    pltpu.sync_copy(s8_hbm.at[pl.ds(base * 8, ROWS_PER_WORKER * 8)], s8_v)

    def chunk(ci, carry):
        pltpu.async_copy(
            tab_hbm.at[idx_v.at[pl.ds(ci * (GATHER_ROWS * K), GATHER_ROWS * K)]],
            buf, sem).wait()
        nbr_iota = lax.iota(jnp.int32, 16)
        for r in range(GATHER_ROWS):
            row = ci * GATHER_ROWS + r
            q = []
            for c in range(SPACE_DIM):
                sn = plsc.load_gather(
                    buf, [nbr_iota + (r * K), jnp.full((16,), PROP_DIM + c, jnp.int32)])
                so = plsc.load_gather(
                    s8_v, [jnp.full((16,), row * 8 + c, jnp.int32)])
                diff = so - sn
                q.append(diff * diff)
            d2 = (q[0] + q[1]) + (q[2] + q[3])
            wv = jnp.exp(-10.0 * d2)
            for cc in range(PROP_DIM // 16):
                ms = [buf[r * K + n, pl.ds(cc * 16, 16)] * wv[n]
                      for n in range(K)]
                acc_m = ms[0]
                for n in range(1, K):
                    acc_m = jnp.maximum(acc_m, ms[n])
                v = list(ms)
                while len(v) > 1:
                    half = len(v) // 2
                    v = [jnp.maximum(v[i] + v[i + half], -BIG)
                         for i in range(half)]
                out_v[row, pl.ds(cc * 16, 16)] = v[0] * (1.0 / K)
                out_v[row, pl.ds(PROP_DIM + cc * 16, 16)] = acc_m
        return carry

    lax.fori_loop(0, N_CHUNKS, chunk, 0)
    pltpu.sync_copy(out_v, agg_hbm.at[pl.ds(base, ROWS_PER_WORKER)])


def _neighbor_agg(idx, s, h):
    idx_p = jnp.pad(idx, ((0, NPAD - N), (0, 0))).reshape(-1)
    tab = jnp.pad(jnp.concatenate([h, s], axis=1),
                  ((0, 0), (0, 128 - PROP_DIM - SPACE_DIM)))
    s8 = jnp.pad(s, ((0, NPAD - N), (0, 8 - SPACE_DIM))).reshape(-1)
    mesh = plsc.VectorSubcoreMesh(core_axis_name="c", subcore_axis_name="s")
    run = functools.partial(
        pl.kernel,
        mesh=mesh,
        compiler_params=pltpu.CompilerParams(needs_layout_passes=False),
        out_type=jax.ShapeDtypeStruct((NPAD, 2 * PROP_DIM), jnp.float32),
        scratch_types=[
            pltpu.VMEM((ROWS_PER_WORKER * K,), jnp.int32),
            pltpu.VMEM((ROWS_PER_WORKER * 8,), jnp.float32),
            pltpu.VMEM((GATHER_ROWS * K, 128), jnp.float32),
            pltpu.VMEM((ROWS_PER_WORKER, 2 * PROP_DIM), jnp.float32),
            pltpu.SemaphoreType.DMA,
        ],
    )(_scagg_body)
    agg = run(idx_p, tab, s8)
    return agg[:N]


def _lin(p, x):
    w, b = p
    return x @ w.T + b


def _bn(gb, x):
    g, b = gb
    mu = jnp.mean(x, axis=0)
    var = jnp.var(x, axis=0)
    return (x - mu) / jnp.sqrt(var + EPS) * g + b


def kernel(x, params):
    x = _bn(params['bn0'], x)
    outs = []
    for blk in params['blocks']:
        g = jnp.mean(x, axis=0, keepdims=True)
        xin = jnp.concatenate([x, jnp.broadcast_to(g, x.shape)], axis=1)
        y = jax.nn.elu(_lin(blk['l0'], xin))
        y = jax.nn.elu(_lin(blk['l1'], y))
        y = _bn(blk['bn1'], y)
        y = jax.nn.elu(_lin(blk['l2'], y))
        h = _lin(blk['lin_h'], y)
        s = _lin(blk['lin_s'], y)
        idx = _knn_stage(s)
        agg = _neighbor_agg(idx, s, h)
        y = y @ blk['out1_W'].T + _lin(blk['out2'], agg)
        y = _bn(blk['bn2'], y)
        outs.append(y)
        x = y
    feat = jax.nn.elu(_lin(params['out'], jnp.concatenate(outs, axis=1)))
    beta = jax.nn.sigmoid(_lin(params['beta'], feat))
    ccoords = _lin(params['ccoords'], feat)
    p = _lin(params['p'], feat)
    vertex = _lin(params['vertex'], feat)
    charge = jax.nn.sigmoid(_lin(params['charge'], feat))
    return jnp.concatenate([beta, ccoords, p, vertex, charge], axis=1)


# speedup vs baseline: 3.3501x; 1.0814x over previous
import functools

import jax
import jax.numpy as jnp
from jax import lax
from jax.experimental import pallas as pl
from jax.experimental.pallas import tpu as pltpu
from jax.experimental.pallas import tpu_sc as plsc

N = 10000
INPUT_DIM = 128
DIM1 = 64
DIM2 = 32
NBLOCKS = 4
SPACE_DIM = 4
PROP_DIM = 64
K = 16
EPS = 1e-5

SC_NC = 2
SC_NS = 16
SC_NW = SC_NC * SC_NS
ROWS_PER_WORKER = 320
NPAD = SC_NW * ROWS_PER_WORKER
GATHER_ROWS = 8
N_CHUNKS = ROWS_PER_WORKER // GATHER_ROWS

KNN_R = 400
BIG = 1e30


def _knn_body(srb_ref, s_ref, n2c_ref, n2r_ref, idx_ref):
    srb = srb_ref[...].astype(jnp.bfloat16)
    s = s_ref[...].astype(jnp.bfloat16)
    g = lax.dot_general(srb, s, (((1,), (1,)), ((), ())),
                        preferred_element_type=jnp.float32)
    d2 = (n2c_ref[...] + n2r_ref[...]) - 2.0 * g
    iota = lax.broadcasted_iota(jnp.int32, (KNN_R, N), 1)
    lane16 = lax.broadcasted_iota(jnp.int32, (KNN_R, K), 1)
    imax = jnp.int32(2**31 - 1)

    def body(k, carry):
        m, j, idxa = carry
        after = (d2 > m) | ((d2 == m) & (iota > j))
        m2 = jnp.min(jnp.where(after, d2, BIG), axis=1, keepdims=True)
        j2 = jnp.min(jnp.where(after & (d2 == m2), iota, imax),
                     axis=1, keepdims=True)
        idxa = jnp.where(lane16 == k, j2, idxa)
        return m2, j2, idxa

    _, _, idxa = lax.fori_loop(
        0, K, body,
        (jnp.full((KNN_R, 1), -BIG, jnp.float32),
         jnp.full((KNN_R, 1), -1, jnp.int32),
         jnp.zeros((KNN_R, K), jnp.int32)))
    idx_ref[...] = idxa


def _knn_stage(s):
    n2 = jnp.sum(s * s, axis=1)
    grid = (N // KNN_R,)
    return pl.pallas_call(
        _knn_body,
        grid=grid,
        in_specs=[
            pl.BlockSpec((KNN_R, SPACE_DIM), lambda i: (i, 0)),
            pl.BlockSpec((N, SPACE_DIM), lambda i: (0, 0)),
            pl.BlockSpec((KNN_R, 1), lambda i: (i, 0)),
            pl.BlockSpec((1, N), lambda i: (0, 0)),
        ],
        out_specs=pl.BlockSpec((KNN_R, K), lambda i: (i, 0)),
        out_shape=jax.ShapeDtypeStruct((N, K), jnp.int32),
    )(s, s, n2[:, None], n2[None, :])


def _scagg_body(idx_hbm, tab_hbm, s8_hbm, agg_hbm, idx_v, s8_v, buf, out_v, sem):
    wid = lax.axis_index("s") * SC_NC + lax.axis_index("c")
    base = wid * ROWS_PER_WORKER
    pltpu.sync_copy(idx_hbm.at[pl.ds(base * K, ROWS_PER_WORKER * K)], idx_v)
    pltpu.sync_copy(s8_hbm.at[pl.ds(base * 8, ROWS_PER_WORKER * 8)], s8_v)

    def chunk(ci, carry):
        pltpu.async_copy(
            tab_hbm.at[idx_v.at[pl.ds(ci * (GATHER_ROWS * K), GATHER_ROWS * K)]],
            buf, sem).wait()
        nbr_iota = lax.iota(jnp.int32, 16)
        for r in range(GATHER_ROWS):
            row = ci * GATHER_ROWS + r
            q = []
            for c in range(SPACE_DIM):
                sn = plsc.load_gather(
                    buf, [nbr_iota + (r * K), jnp.full((16,), PROP_DIM + c, jnp.int32)])
                so = plsc.load_gather(
                    s8_v, [jnp.full((16,), row * 8 + c, jnp.int32)])
                diff = so - sn
                q.append(diff * diff)
            d2 = (q[0] + q[1]) + (q[2] + q[3])
            wv = jnp.exp(-10.0 * d2)
            for cc in range(PROP_DIM // 16):
                ms = [buf[r * K + n, pl.ds(cc * 16, 16)] * wv[n]
                      for n in range(K)]
                acc_m = ms[0]
                for n in range(1, K):
                    acc_m = jnp.maximum(acc_m, ms[n])
                v = list(ms)
                while len(v) > 1:
                    half = len(v) // 2
                    v = [jnp.maximum(v[i] + v[i + half], -BIG)
                         for i in range(half)]
                out_v[row, pl.ds(cc * 16, 16)] = v[0] * (1.0 / K)
                out_v[row, pl.ds(PROP_DIM + cc * 16, 16)] = acc_m
        return carry

    lax.fori_loop(0, N_CHUNKS, chunk, 0)
    pltpu.sync_copy(out_v, agg_hbm.at[pl.ds(base, ROWS_PER_WORKER)])


def _neighbor_agg(idx, s, h):
    idx_p = jnp.pad(idx, ((0, NPAD - N), (0, 0))).reshape(-1)
    tab = jnp.pad(jnp.concatenate([h, s], axis=1),
                  ((0, 0), (0, 128 - PROP_DIM - SPACE_DIM)))
    s8 = jnp.pad(s, ((0, NPAD - N), (0, 8 - SPACE_DIM))).reshape(-1)
    mesh = plsc.VectorSubcoreMesh(core_axis_name="c", subcore_axis_name="s")
    run = functools.partial(
        pl.kernel,
        mesh=mesh,
        compiler_params=pltpu.CompilerParams(needs_layout_passes=False),
        out_type=jax.ShapeDtypeStruct((NPAD, 2 * PROP_DIM), jnp.float32),
        scratch_types=[
            pltpu.VMEM((ROWS_PER_WORKER * K,), jnp.int32),
            pltpu.VMEM((ROWS_PER_WORKER * 8,), jnp.float32),
            pltpu.VMEM((GATHER_ROWS * K, 128), jnp.float32),
            pltpu.VMEM((ROWS_PER_WORKER, 2 * PROP_DIM), jnp.float32),
            pltpu.SemaphoreType.DMA,
        ],
    )(_scagg_body)
    agg = run(idx_p, tab, s8)
    return agg[:N]


def _lin(p, x):
    w, b = p
    return x @ w.T + b


def _bn(gb, x):
    g, b = gb
    mu = jnp.mean(x, axis=0)
    var = jnp.var(x, axis=0)
    return (x - mu) / jnp.sqrt(var + EPS) * g + b


def kernel(x, params):
    x = _bn(params['bn0'], x)
    outs = []
    for blk in params['blocks']:
        g = jnp.mean(x, axis=0, keepdims=True)
        xin = jnp.concatenate([x, jnp.broadcast_to(g, x.shape)], axis=1)
        y = jax.nn.elu(_lin(blk['l0'], xin))
        y = jax.nn.elu(_lin(blk['l1'], y))
        y = _bn(blk['bn1'], y)
        y = jax.nn.elu(_lin(blk['l2'], y))
        h = _lin(blk['lin_h'], y)
        s = _lin(blk['lin_s'], y)
        idx = _knn_stage(s)
        agg = _neighbor_agg(idx, s, h)
        y = y @ blk['out1_W'].T + _lin(blk['out2'], agg)
        y = _bn(blk['bn2'], y)
        outs.append(y)
        x = y
    feat = jax.nn.elu(_lin(params['out'], jnp.concatenate(outs, axis=1)))
    beta = jax.nn.sigmoid(_lin(params['beta'], feat))
    ccoords = _lin(params['ccoords'], feat)
    p = _lin(params['p'], feat)
    vertex = _lin(params['vertex'], feat)
    charge = jax.nn.sigmoid(_lin(params['charge'], feat))
    return jnp.concatenate([beta, ccoords, p, vertex, charge], axis=1)
